# baseline (speedup 1.0000x reference)
import functools

import jax
import jax.numpy as jnp
from jax import lax
from jax.experimental import pallas as pl
from jax.experimental.pallas import tpu as pltpu
from jax.experimental.pallas import tpu_sc as plsc

N_NODES = 10000
N_PAD = 10240
N_EDGES = 320000
HID = 128
NCL = 15
N_SC = 2
N_SUB = 16
N_WORK = N_SC * N_SUB
CHUNK = 80
EDGES_PER_WORKER = N_EDGES // N_WORK
CHUNKS_PER_WORKER = EDGES_PER_WORKER // CHUNK
ROWS_PER_SUB = N_PAD // N_SUB
ZROWS = 128
HHID = HID // 2
AGG_CHUNKS = N_EDGES // N_SUB // CHUNK
NBUF = 5

RB = 1000
GRID = N_NODES // RB


def _fill_zeros(buf, rows, cols):
    z = jnp.zeros((16,), jnp.float32)

    def body(r, _):
        for k in range(cols // 16):
            buf[r, pl.ds(k * 16, 16)] = z
        return 0

    lax.fori_loop(0, rows, body, 0)


def _sc_mesh():
    return plsc.VectorSubcoreMesh(
        core_axis_name="c", subcore_axis_name="s",
        num_cores=N_SC, num_subcores=N_SUB)


@functools.lru_cache(maxsize=None)
def _agg_kernel():
    @functools.partial(
        pl.kernel,
        mesh=_sc_mesh(),
        compiler_params=pltpu.CompilerParams(use_tc_tiling_on_sc=False),
        out_type=jax.ShapeDtypeStruct((N_SC, N_PAD, HHID), jnp.float32),
        scratch_types=[
            pltpu.VMEM((AGG_CHUNKS, CHUNK), jnp.int32),
            pltpu.VMEM((AGG_CHUNKS, CHUNK), jnp.int32),
            [pltpu.VMEM((CHUNK, HHID), jnp.float32)] * NBUF,
            pltpu.VMEM((ZROWS, HHID), jnp.float32),
            pltpu.VMEM_SHARED((N_PAD, HHID), jnp.float32),
            [pltpu.SemaphoreType.DMA] * NBUF,
        ],
    )
    def agg(g2, e3, out, src_v, dst_v, rows, zbuf, acc, gsem):
        c = lax.axis_index("c")
        s = lax.axis_index("s")
        _fill_zeros(zbuf, ZROWS, HHID)
        for k in range(ROWS_PER_SUB // ZROWS):
            pltpu.sync_copy(zbuf, acc.at[pl.ds(s * ROWS_PER_SUB + k * ZROWS, ZROWS)])
        plsc.subcore_barrier()
        pltpu.sync_copy(e3.at[0, s], src_v)
        pltpu.sync_copy(e3.at[1, s], dst_v)
        g_half = g2.at[c]

        for b in range(NBUF - 1):
            pltpu.async_copy(g_half.at[src_v.at[b]], rows[b], gsem[b])

        def ring(jr, _):
            for b in range(NBUF):
                j = jr * NBUF + b
                nxt = j + NBUF - 1
                bn = (b + NBUF - 1) % NBUF

                @pl.when(nxt < AGG_CHUNKS)
                def _():
                    pltpu.async_copy(g_half.at[src_v.at[nxt]], rows[bn], gsem[bn])

                pltpu.make_async_copy(g_half.at[src_v.at[j]], rows[b],
                                      gsem[b]).wait()
                pltpu.sync_copy(rows[b], acc.at[dst_v.at[j]], add=True)
            return 0

        lax.fori_loop(0, AGG_CHUNKS // NBUF, ring, 0)
        plsc.subcore_barrier()
        pltpu.sync_copy(acc.at[pl.ds(s * ROWS_PER_SUB, ROWS_PER_SUB)],
                        out.at[c, pl.ds(s * ROWS_PER_SUB, ROWS_PER_SUB)])

    return agg


@functools.lru_cache(maxsize=None)
def _deg_kernel():
    @functools.partial(
        pl.kernel,
        mesh=_sc_mesh(),
        compiler_params=pltpu.CompilerParams(use_tc_tiling_on_sc=False),
        out_type=jax.ShapeDtypeStruct((N_SC, N_PAD, 16), jnp.float32),
        scratch_types=[
            pltpu.VMEM((CHUNKS_PER_WORKER, CHUNK), jnp.int32),
            pltpu.VMEM((CHUNK, 16), jnp.float32),
            pltpu.VMEM((ZROWS, 16), jnp.float32),
            pltpu.VMEM_SHARED((N_PAD, 16), jnp.float32),
        ],
    )
    def deg(e4, out, dst_v, ones_v, zbuf, acc):
        c = lax.axis_index("c")
        s = lax.axis_index("s")
        wid = s * N_SC + c
        _fill_zeros(zbuf, ZROWS, 16)
        for k in range(ROWS_PER_SUB // ZROWS):
            pltpu.sync_copy(zbuf, acc.at[pl.ds(s * ROWS_PER_SUB + k * ZROWS, ZROWS)])
        e0 = jnp.where(lax.iota(jnp.int32, 16) == 0, 1.0, 0.0).astype(jnp.float32)

        def fill_ones(i, _):
            ones_v[i, pl.ds(0, 16)] = e0
            return 0

        lax.fori_loop(0, CHUNK, fill_ones, 0)
        plsc.subcore_barrier()
        pltpu.sync_copy(e4.at[1, wid], dst_v)

        def chunk(j, _):
            pltpu.sync_copy(ones_v, acc.at[dst_v.at[j]], add=True)
            return 0

        lax.fori_loop(0, CHUNKS_PER_WORKER, chunk, 0)
        plsc.subcore_barrier()
        pltpu.sync_copy(acc.at[pl.ds(s * ROWS_PER_SUB, ROWS_PER_SUB)],
                        out.at[c, pl.ds(s * ROWS_PER_SUB, ROWS_PER_SUB)])

    return deg


def _rowblk(last):
    return pl.BlockSpec((RB, last), lambda i: (i, 0))


def _colblk2(last):
    return pl.BlockSpec((2, RB, last), lambda i: (0, i, 0))


def _full(*shape):
    return pl.BlockSpec(shape, lambda i: (0,) * len(shape))


def _sigmoid(x):
    return 1.0 / (1.0 + jnp.exp(-x))


@functools.lru_cache(maxsize=None)
def _enc_kernel():
    def body(x_ref, w1_ref, b1_ref, w2_ref, b2_ref, h_ref):
        t = jax.nn.relu(
            jnp.dot(x_ref[...], w1_ref[...], preferred_element_type=jnp.float32)
            + b1_ref[...])
        h_ref[...] = (
            jnp.dot(t, w2_ref[...], preferred_element_type=jnp.float32)
            + b2_ref[...])

    return pl.pallas_call(
        body,
        grid=(GRID,),
        in_specs=[_rowblk(HID), _full(HID, HID), _full(1, HID),
                  _full(HID, HID), _full(1, HID)],
        out_specs=_rowblk(HID),
        out_shape=jax.ShapeDtypeStruct((N_NODES, HID), jnp.float32),
    )


@functools.lru_cache(maxsize=None)
def _prep0_kernel():
    def body(h_ref, dp_ref, w_ref, g2_ref, dis_ref):
        deg = dp_ref[0, :, 0:1] + dp_ref[1, :, 0:1] + 1.0
        dis = lax.rsqrt(deg)
        dis_ref[...] = dis
        g = jnp.dot(h_ref[...], w_ref[...],
                    preferred_element_type=jnp.float32) * dis
        g2_ref[0] = g[:, :HHID]
        g2_ref[1] = g[:, HHID:]

    return pl.pallas_call(
        body,
        grid=(GRID,),
        in_specs=[_rowblk(HID), _colblk2(16), _full(HID, HID)],
        out_specs=[_colblk2(HHID), _rowblk(1)],
        out_shape=[jax.ShapeDtypeStruct((N_SC, N_NODES, HHID), jnp.float32),
                   jax.ShapeDtypeStruct((N_NODES, 1), jnp.float32)],
    )


@functools.lru_cache(maxsize=None)
def _layer_kernel(residual, last):
    def body(*refs):
        if residual:
            p_ref, g2_ref, dis_ref, b_ref, hp_ref, w_ref = refs[:6]
            out_refs = refs[6:]
        else:
            p_ref, g2_ref, dis_ref, b_ref, w_ref = refs[:5]
            out_refs = refs[5:]
        dis = dis_ref[...]
        asum = jnp.concatenate(
            [p_ref[0] + g2_ref[0], p_ref[1] + g2_ref[1]], axis=-1)
        hn = jax.nn.relu(dis * asum + b_ref[...])
        h = hp_ref[...] + hn if residual else hn
        h_ref = out_refs[0]
        h_ref[...] = h
        if last:
            cs_ref = out_refs[1]

            @pl.when(pl.program_id(0) == 0)
            def _():
                cs_ref[...] = jnp.zeros_like(cs_ref)

            cs_ref[...] += jnp.sum(h, axis=0, keepdims=True)
        else:
            g2n_ref = out_refs[1]
            gn = jnp.dot(h, w_ref[...], preferred_element_type=jnp.float32) * dis
            g2n_ref[0] = gn[:, :HHID]
            g2n_ref[1] = gn[:, HHID:]

    in_specs = [_colblk2(HHID), _colblk2(HHID), _rowblk(1), _full(1, HID)]
    if residual:
        in_specs.append(_rowblk(HID))
    in_specs.append(_full(HID, HID))
    out_specs = [_rowblk(HID)]
    out_shape = [jax.ShapeDtypeStruct((N_NODES, HID), jnp.float32)]
    if last:
        out_specs.append(pl.BlockSpec((1, HID), lambda i: (0, 0)))
        out_shape.append(jax.ShapeDtypeStruct((1, HID), jnp.float32))
    else:
        out_specs.append(_colblk2(HHID))
        out_shape.append(jax.ShapeDtypeStruct((N_SC, N_NODES, HHID), jnp.float32))

    return pl.pallas_call(
        body,
        grid=(GRID,),
        in_specs=in_specs,
        out_specs=out_specs,
        out_shape=out_shape,
    )


@functools.lru_cache(maxsize=None)
def _heads_kernel():
    def body(h_ref, cs_ref, cw1_ref, cb1_ref, cw2_ref, cb2_ref,
             sw1a_ref, sw1b_ref, sb1_ref, sw2_ref, sb2_ref,
             lg_ref, pr_ref, so_ref):
        h = h_ref[...]
        t = jax.nn.relu(
            jnp.dot(h, cw1_ref[...], preferred_element_type=jnp.float32)
            + cb1_ref[...])
        logits = (jnp.dot(t, cw2_ref[...], preferred_element_type=jnp.float32)
                  + cb2_ref[...])
        lg_ref[...] = logits
        m = jnp.max(logits, axis=-1, keepdims=True)
        e = jnp.exp(logits - m)
        pr_ref[...] = e / jnp.sum(e, axis=-1, keepdims=True)
        mean = cs_ref[...] * (1.0 / N_NODES)
        mc = jnp.dot(mean, sw1b_ref[...],
                     preferred_element_type=jnp.float32) + sb1_ref[...]
        u = jax.nn.relu(
            jnp.dot(h, sw1a_ref[...], preferred_element_type=jnp.float32) + mc)
        so_ref[...] = _sigmoid(
            jnp.dot(u, sw2_ref[...], preferred_element_type=jnp.float32)
            + sb2_ref[...])

    return pl.pallas_call(
        body,
        grid=(GRID,),
        in_specs=[_rowblk(HID), _full(1, HID),
                  _full(HID, HHID), _full(1, HHID), _full(HHID, NCL), _full(1, NCL),
                  _full(HID, HID), _full(HID, HID), _full(1, HID),
                  _full(HID, 1), _full(1, 1)],
        out_specs=[_rowblk(NCL), _rowblk(NCL), _rowblk(1)],
        out_shape=[jax.ShapeDtypeStruct((N_NODES, NCL), jnp.float32),
                   jax.ShapeDtypeStruct((N_NODES, NCL), jnp.float32),
                   jax.ShapeDtypeStruct((N_NODES, 1), jnp.float32)],
    )


def kernel(x, edge_index, enc_w1, enc_b1, enc_w2, enc_b2,
           gcn_w0, gcn_b0, gcn_w1, gcn_b1, gcn_w2, gcn_b2,
           cl_w1, cl_b1, cl_w2, cl_b2, so_w1, so_b1, so_w2, so_b2):
    e4 = edge_index.reshape(2, N_WORK, CHUNKS_PER_WORKER, CHUNK)
    e3 = edge_index.reshape(2, N_SUB, AGG_CHUNKS, CHUNK)

    deg_parts = _deg_kernel()(e4)
    h0 = _enc_kernel()(x, enc_w1, enc_b1.reshape(1, HID),
                       enc_w2, enc_b2.reshape(1, HID))
    g2, dis = _prep0_kernel()(h0, deg_parts, gcn_w0)

    parts = _agg_kernel()(g2, e3)
    h1, g2 = _layer_kernel(False, False)(
        parts, g2, dis, gcn_b0.reshape(1, HID), gcn_w1)

    parts = _agg_kernel()(g2, e3)
    h2, g2 = _layer_kernel(True, False)(
        parts, g2, dis, gcn_b1.reshape(1, HID), h1, gcn_w2)

    parts = _agg_kernel()(g2, e3)
    h3, colsum = _layer_kernel(True, True)(
        parts, g2, dis, gcn_b2.reshape(1, HID), h2, gcn_w2)

    logits, probs, solar = _heads_kernel()(
        h3, colsum,
        cl_w1, cl_b1.reshape(1, HHID), cl_w2, cl_b2.reshape(1, NCL),
        so_w1[:HID], so_w1[HID:], so_b1.reshape(1, HID),
        so_w2, so_b2.reshape(1, 1))

    return (logits, probs, solar[:, 0], h3)


# speedup vs baseline: 1.0214x; 1.0214x over previous
import functools

import jax
import jax.numpy as jnp
from jax import lax
from jax.experimental import pallas as pl
from jax.experimental.pallas import tpu as pltpu
from jax.experimental.pallas import tpu_sc as plsc

N_NODES = 10000
N_PAD = 10240
N_EDGES = 320000
HID = 128
NCL = 15
N_SC = 2
N_SUB = 16
N_WORK = N_SC * N_SUB
CHUNK = 80
EDGES_PER_WORKER = N_EDGES // N_WORK
CHUNKS_PER_WORKER = EDGES_PER_WORKER // CHUNK
ROWS_PER_SUB = N_PAD // N_SUB
ZROWS = 128
HHID = HID // 2
AGG_CHUNKS = N_EDGES // N_SUB // CHUNK
NBUF = 5

RB = 2000
GRID = N_NODES // RB


def _fill_zeros(buf, rows, cols):
    z = jnp.zeros((16,), jnp.float32)

    def body(r, _):
        for k in range(cols // 16):
            buf[r, pl.ds(k * 16, 16)] = z
        return 0

    lax.fori_loop(0, rows, body, 0)


def _sc_mesh():
    return plsc.VectorSubcoreMesh(
        core_axis_name="c", subcore_axis_name="s",
        num_cores=N_SC, num_subcores=N_SUB)


@functools.lru_cache(maxsize=None)
def _agg_kernel():
    @functools.partial(
        pl.kernel,
        mesh=_sc_mesh(),
        compiler_params=pltpu.CompilerParams(use_tc_tiling_on_sc=False),
        out_type=jax.ShapeDtypeStruct((N_SC, N_PAD, HHID), jnp.float32),
        scratch_types=[
            pltpu.VMEM((AGG_CHUNKS, CHUNK), jnp.int32),
            pltpu.VMEM((AGG_CHUNKS, CHUNK), jnp.int32),
            [pltpu.VMEM((CHUNK, HHID), jnp.float32)] * NBUF,
            pltpu.VMEM((ZROWS, HHID), jnp.float32),
            pltpu.VMEM_SHARED((N_PAD, HHID), jnp.float32),
            [pltpu.SemaphoreType.DMA] * NBUF,
        ],
    )
    def agg(g2, e3, out, src_v, dst_v, rows, zbuf, acc, gsem):
        c = lax.axis_index("c")
        s = lax.axis_index("s")
        _fill_zeros(zbuf, ZROWS, HHID)
        for k in range(ROWS_PER_SUB // ZROWS):
            pltpu.sync_copy(zbuf, acc.at[pl.ds(s * ROWS_PER_SUB + k * ZROWS, ZROWS)])
        plsc.subcore_barrier()
        pltpu.sync_copy(e3.at[0, s], src_v)
        pltpu.sync_copy(e3.at[1, s], dst_v)
        g_half = g2.at[c]

        for b in range(NBUF - 1):
            pltpu.async_copy(g_half.at[src_v.at[b]], rows[b], gsem[b])

        def ring(jr, _):
            for b in range(NBUF):
                j = jr * NBUF + b
                nxt = j + NBUF - 1
                bn = (b + NBUF - 1) % NBUF

                @pl.when(nxt < AGG_CHUNKS)
                def _():
                    pltpu.async_copy(g_half.at[src_v.at[nxt]], rows[bn], gsem[bn])

                pltpu.make_async_copy(g_half.at[src_v.at[j]], rows[b],
                                      gsem[b]).wait()
                pltpu.sync_copy(rows[b], acc.at[dst_v.at[j]], add=True)
            return 0

        lax.fori_loop(0, AGG_CHUNKS // NBUF, ring, 0)
        plsc.subcore_barrier()
        pltpu.sync_copy(acc.at[pl.ds(s * ROWS_PER_SUB, ROWS_PER_SUB)],
                        out.at[c, pl.ds(s * ROWS_PER_SUB, ROWS_PER_SUB)])

    return agg


@functools.lru_cache(maxsize=None)
def _deg_kernel():
    @functools.partial(
        pl.kernel,
        mesh=_sc_mesh(),
        compiler_params=pltpu.CompilerParams(use_tc_tiling_on_sc=False),
        out_type=jax.ShapeDtypeStruct((N_SC, N_PAD, 16), jnp.float32),
        scratch_types=[
            pltpu.VMEM((CHUNKS_PER_WORKER, CHUNK), jnp.int32),
            pltpu.VMEM((CHUNK, 16), jnp.float32),
            pltpu.VMEM((ZROWS, 16), jnp.float32),
            pltpu.VMEM_SHARED((N_PAD, 16), jnp.float32),
        ],
    )
    def deg(e4, out, dst_v, ones_v, zbuf, acc):
        c = lax.axis_index("c")
        s = lax.axis_index("s")
        wid = s * N_SC + c
        _fill_zeros(zbuf, ZROWS, 16)
        for k in range(ROWS_PER_SUB // ZROWS):
            pltpu.sync_copy(zbuf, acc.at[pl.ds(s * ROWS_PER_SUB + k * ZROWS, ZROWS)])
        e0 = jnp.where(lax.iota(jnp.int32, 16) == 0, 1.0, 0.0).astype(jnp.float32)

        def fill_ones(i, _):
            ones_v[i, pl.ds(0, 16)] = e0
            return 0

        lax.fori_loop(0, CHUNK, fill_ones, 0)
        plsc.subcore_barrier()
        pltpu.sync_copy(e4.at[1, wid], dst_v)

        def chunk(j, _):
            pltpu.sync_copy(ones_v, acc.at[dst_v.at[j]], add=True)
            return 0

        lax.fori_loop(0, CHUNKS_PER_WORKER, chunk, 0)
        plsc.subcore_barrier()
        pltpu.sync_copy(acc.at[pl.ds(s * ROWS_PER_SUB, ROWS_PER_SUB)],
                        out.at[c, pl.ds(s * ROWS_PER_SUB, ROWS_PER_SUB)])

    return deg


def _rowblk(last):
    return pl.BlockSpec((RB, last), lambda i: (i, 0))


def _colblk2(last):
    return pl.BlockSpec((2, RB, last), lambda i: (0, i, 0))


def _full(*shape):
    return pl.BlockSpec(shape, lambda i: (0,) * len(shape))


def _sigmoid(x):
    return 1.0 / (1.0 + jnp.exp(-x))


@functools.lru_cache(maxsize=None)
def _enc_kernel():
    def body(x_ref, w1_ref, b1_ref, w2_ref, b2_ref, h_ref):
        t = jax.nn.relu(
            jnp.dot(x_ref[...], w1_ref[...], preferred_element_type=jnp.float32)
            + b1_ref[...])
        h_ref[...] = (
            jnp.dot(t, w2_ref[...], preferred_element_type=jnp.float32)
            + b2_ref[...])

    return pl.pallas_call(
        body,
        grid=(GRID,),
        in_specs=[_rowblk(HID), _full(HID, HID), _full(1, HID),
                  _full(HID, HID), _full(1, HID)],
        out_specs=_rowblk(HID),
        out_shape=jax.ShapeDtypeStruct((N_NODES, HID), jnp.float32),
    )


@functools.lru_cache(maxsize=None)
def _prep0_kernel():
    def body(h_ref, dp_ref, w_ref, g2_ref, dis_ref):
        deg = dp_ref[0, :, 0:1] + dp_ref[1, :, 0:1] + 1.0
        dis = lax.rsqrt(deg)
        dis_ref[...] = dis
        g = jnp.dot(h_ref[...], w_ref[...],
                    preferred_element_type=jnp.float32) * dis
        g2_ref[0] = g[:, :HHID]
        g2_ref[1] = g[:, HHID:]

    return pl.pallas_call(
        body,
        grid=(GRID,),
        in_specs=[_rowblk(HID), _colblk2(16), _full(HID, HID)],
        out_specs=[_colblk2(HHID), _rowblk(1)],
        out_shape=[jax.ShapeDtypeStruct((N_SC, N_NODES, HHID), jnp.float32),
                   jax.ShapeDtypeStruct((N_NODES, 1), jnp.float32)],
    )


@functools.lru_cache(maxsize=None)
def _layer_kernel(residual, last):
    def body(*refs):
        if residual:
            p_ref, g2_ref, dis_ref, b_ref, hp_ref, w_ref = refs[:6]
            out_refs = refs[6:]
        else:
            p_ref, g2_ref, dis_ref, b_ref, w_ref = refs[:5]
            out_refs = refs[5:]
        dis = dis_ref[...]
        asum = jnp.concatenate(
            [p_ref[0] + g2_ref[0], p_ref[1] + g2_ref[1]], axis=-1)
        hn = jax.nn.relu(dis * asum + b_ref[...])
        h = hp_ref[...] + hn if residual else hn
        h_ref = out_refs[0]
        h_ref[...] = h
        if last:
            cs_ref = out_refs[1]

            @pl.when(pl.program_id(0) == 0)
            def _():
                cs_ref[...] = jnp.zeros_like(cs_ref)

            cs_ref[...] += jnp.sum(h, axis=0, keepdims=True)
        else:
            g2n_ref = out_refs[1]
            gn = jnp.dot(h, w_ref[...], preferred_element_type=jnp.float32) * dis
            g2n_ref[0] = gn[:, :HHID]
            g2n_ref[1] = gn[:, HHID:]

    in_specs = [_colblk2(HHID), _colblk2(HHID), _rowblk(1), _full(1, HID)]
    if residual:
        in_specs.append(_rowblk(HID))
    in_specs.append(_full(HID, HID))
    out_specs = [_rowblk(HID)]
    out_shape = [jax.ShapeDtypeStruct((N_NODES, HID), jnp.float32)]
    if last:
        out_specs.append(pl.BlockSpec((1, HID), lambda i: (0, 0)))
        out_shape.append(jax.ShapeDtypeStruct((1, HID), jnp.float32))
    else:
        out_specs.append(_colblk2(HHID))
        out_shape.append(jax.ShapeDtypeStruct((N_SC, N_NODES, HHID), jnp.float32))

    return pl.pallas_call(
        body,
        grid=(GRID,),
        in_specs=in_specs,
        out_specs=out_specs,
        out_shape=out_shape,
    )


@functools.lru_cache(maxsize=None)
def _heads_kernel():
    def body(h_ref, cs_ref, cw1_ref, cb1_ref, cw2_ref, cb2_ref,
             sw1a_ref, sw1b_ref, sb1_ref, sw2_ref, sb2_ref,
             lg_ref, pr_ref, so_ref):
        h = h_ref[...]
        t = jax.nn.relu(
            jnp.dot(h, cw1_ref[...], preferred_element_type=jnp.float32)
            + cb1_ref[...])
        logits = (jnp.dot(t, cw2_ref[...], preferred_element_type=jnp.float32)
                  + cb2_ref[...])
        lg_ref[...] = logits
        m = jnp.max(logits, axis=-1, keepdims=True)
        e = jnp.exp(logits - m)
        pr_ref[...] = e / jnp.sum(e, axis=-1, keepdims=True)
        mean = cs_ref[...] * (1.0 / N_NODES)
        mc = jnp.dot(mean, sw1b_ref[...],
                     preferred_element_type=jnp.float32) + sb1_ref[...]
        u = jax.nn.relu(
            jnp.dot(h, sw1a_ref[...], preferred_element_type=jnp.float32) + mc)
        so_ref[...] = _sigmoid(
            jnp.dot(u, sw2_ref[...], preferred_element_type=jnp.float32)
            + sb2_ref[...])

    return pl.pallas_call(
        body,
        grid=(GRID,),
        in_specs=[_rowblk(HID), _full(1, HID),
                  _full(HID, HHID), _full(1, HHID), _full(HHID, NCL), _full(1, NCL),
                  _full(HID, HID), _full(HID, HID), _full(1, HID),
                  _full(HID, 1), _full(1, 1)],
        out_specs=[_rowblk(NCL), _rowblk(NCL), _rowblk(1)],
        out_shape=[jax.ShapeDtypeStruct((N_NODES, NCL), jnp.float32),
                   jax.ShapeDtypeStruct((N_NODES, NCL), jnp.float32),
                   jax.ShapeDtypeStruct((N_NODES, 1), jnp.float32)],
    )


def kernel(x, edge_index, enc_w1, enc_b1, enc_w2, enc_b2,
           gcn_w0, gcn_b0, gcn_w1, gcn_b1, gcn_w2, gcn_b2,
           cl_w1, cl_b1, cl_w2, cl_b2, so_w1, so_b1, so_w2, so_b2):
    e4 = edge_index.reshape(2, N_WORK, CHUNKS_PER_WORKER, CHUNK)
    e3 = edge_index.reshape(2, N_SUB, AGG_CHUNKS, CHUNK)

    deg_parts = _deg_kernel()(e4)
    h0 = _enc_kernel()(x, enc_w1, enc_b1.reshape(1, HID),
                       enc_w2, enc_b2.reshape(1, HID))
    g2, dis = _prep0_kernel()(h0, deg_parts, gcn_w0)

    parts = _agg_kernel()(g2, e3)
    h1, g2 = _layer_kernel(False, False)(
        parts, g2, dis, gcn_b0.reshape(1, HID), gcn_w1)

    parts = _agg_kernel()(g2, e3)
    h2, g2 = _layer_kernel(True, False)(
        parts, g2, dis, gcn_b1.reshape(1, HID), h1, gcn_w2)

    parts = _agg_kernel()(g2, e3)
    h3, colsum = _layer_kernel(True, True)(
        parts, g2, dis, gcn_b2.reshape(1, HID), h2, gcn_w2)

    logits, probs, solar = _heads_kernel()(
        h3, colsum,
        cl_w1, cl_b1.reshape(1, HHID), cl_w2, cl_b2.reshape(1, NCL),
        so_w1[:HID], so_w1[HID:], so_b1.reshape(1, HID),
        so_w2, so_b2.reshape(1, 1))

    return (logits, probs, solar[:, 0], h3)
